# baseline (speedup 1.0000x reference)
import functools

import jax
import jax.numpy as jnp
from jax import lax
from jax.experimental import pallas as pl
from jax.experimental.pallas import tpu as pltpu
from jax.experimental.pallas import tpu_sc as plsc

EMBED = 64
LANES = 16
NC = 2
NS = 16
NW = NC * NS

STEP = 512
TOK_BITS = 20
TOK_MASK = (1 << TOK_BITS) - 1


def _build(ntok, n_pos_rows):
    rows_per_w = ntok // NW
    nsteps = rows_per_w // STEP
    pos_words = n_pos_rows * EMBED
    mesh = plsc.VectorSubcoreMesh(core_axis_name="c", subcore_axis_name="s")

    @functools.partial(
        pl.kernel,
        out_type=jax.ShapeDtypeStruct((ntok, EMBED), jnp.float32),
        mesh=mesh,
        compiler_params=pltpu.CompilerParams(use_tc_tiling_on_sc=False,
                                             needs_layout_passes=False),
        scratch_types=[
            pltpu.VMEM((rows_per_w,), jnp.int32),
            pltpu.VMEM((pos_words,), jnp.float32),
            pltpu.VMEM((2, STEP), jnp.int32),
            pltpu.VMEM((STEP, EMBED), jnp.float32),
            pltpu.VMEM((STEP, EMBED), jnp.float32),
            pltpu.SemaphoreType.DMA,
            pltpu.SemaphoreType.DMA,
            pltpu.SemaphoreType.DMA,
            pltpu.SemaphoreType.DMA,
        ],
    )
    def emb_kernel(packed_hbm, table_hbm, pos_hbm, out_hbm,
                   packed_v, pos_v, tok_v, rows0, rows1,
                   sgt0, sgt1, so0, so1):
        wid = lax.axis_index("s") * NC + lax.axis_index("c")
        w_base = wid * rows_per_w
        rows = (rows0, rows1)
        sgt = (sgt0, sgt1)
        so = (so0, so1)

        pltpu.async_copy(packed_hbm.at[pl.ds(w_base, rows_per_w)],
                         packed_v, sgt0)
        pltpu.async_copy(pos_hbm, pos_v, sgt1)
        pltpu.make_async_copy(packed_hbm.at[pl.ds(w_base, rows_per_w)],
                              packed_v, sgt0).wait()
        pltpu.make_async_copy(pos_hbm, pos_v, sgt1).wait()

        def issue_gather(st, b):
            off = pl.multiple_of(st * STEP, 8)

            @plsc.parallel_loop(0, STEP // LANES, unroll=4)
            def _unpack(g):
                sl = pl.ds(off + g * LANES, LANES)
                tok_v[b, pl.ds(g * LANES, LANES)] = packed_v[sl] & TOK_MASK

            pltpu.async_copy(table_hbm.at[tok_v.at[b]], rows[b], sgt[b])

        def wait_gather(b):
            pltpu.make_async_copy(table_hbm.at[tok_v.at[b]], rows[b],
                                  sgt[b]).wait()

        def wait_out(b):
            pltpu.make_async_copy(rows[b], out_hbm.at[pl.ds(0, STEP)],
                                  so[b]).wait()

        issue_gather(0, 0)
        lane = lax.iota(jnp.int32, LANES)

        def pair_body(j, carry):
            for b in range(2):
                st = 2 * j + b
                nb = 1 - b

                @pl.when(st + 1 < nsteps)
                def _issue_next():
                    @pl.when(st >= 1)
                    def _drain_out():
                        wait_out(nb)
                    issue_gather(st + 1, nb)

                wait_gather(b)
                off = pl.multiple_of(st * STEP, 8)

                @plsc.parallel_loop(0, STEP // LANES, unroll=2)
                def _pos_add(g):
                    pk = packed_v[pl.ds(off + g * LANES, LANES)]
                    paddr = (pk >> TOK_BITS) * EMBED
                    ridx = g * LANES + lane
                    for c in range(EMBED):
                        cidx = jnp.full((LANES,), c, jnp.int32)
                        pv = plsc.load_gather(pos_v, [paddr + c])
                        plsc.addupdate_scatter(rows[b], [ridx, cidx], pv)

                base = pl.multiple_of(w_base + st * STEP, 8)
                pltpu.async_copy(rows[b], out_hbm.at[pl.ds(base, STEP)], so[b])
            return carry

        lax.fori_loop(0, nsteps // 2, pair_body, 0)
        wait_out(0)
        wait_out(1)

    return emb_kernel


def kernel(x, padding_mask, table, pos_enc):
    b, s = x.shape
    ntok = b * s
    n_pos_rows = pos_enc.shape[0] + 1
    s_ids = jnp.arange(s, dtype=jnp.int32)[None, :]
    pp = jnp.where(padding_mask, jnp.int32(n_pos_rows - 1), s_ids)
    packed = (x.astype(jnp.int32) | (pp << TOK_BITS)).reshape(ntok)
    pos_flat = jnp.concatenate(
        [pos_enc.astype(jnp.float32),
         jnp.zeros((1, pos_enc.shape[1]), jnp.float32)], axis=0).reshape(-1)
    out = _build(ntok, n_pos_rows)(packed, table, pos_flat)
    return out.reshape(b, s, EMBED)


# speedup vs baseline: 1.2816x; 1.2816x over previous
import functools

import jax
import jax.numpy as jnp
from jax import lax
from jax.experimental import pallas as pl
from jax.experimental.pallas import tpu as pltpu
from jax.experimental.pallas import tpu_sc as plsc

EMBED = 64
LANES = 16
NC = 2
NS = 16
NW = NC * NS

SEQ = 200
ROWS_STEP = 2
STEP = ROWS_STEP * SEQ
TOK_BITS = 20
TOK_MASK = (1 << TOK_BITS) - 1
POS_REP = LANES


def _build(batch, n_pos_rows):
    rows_per_w = batch // NW * SEQ
    nsteps = batch // NW // ROWS_STEP
    mesh = plsc.VectorSubcoreMesh(core_axis_name="c", subcore_axis_name="s")

    @functools.partial(
        pl.kernel,
        out_type=jax.ShapeDtypeStruct((batch, SEQ, EMBED), jnp.float32),
        mesh=mesh,
        compiler_params=pltpu.CompilerParams(use_tc_tiling_on_sc=False,
                                             needs_layout_passes=False),
        scratch_types=[
            pltpu.VMEM((rows_per_w,), jnp.int32),
            pltpu.VMEM((2, STEP), jnp.int32),
            pltpu.VMEM((2, STEP), jnp.int32),
            pltpu.VMEM((2, ROWS_STEP, SEQ, EMBED), jnp.float32),
            pltpu.VMEM((2, ROWS_STEP, SEQ, EMBED), jnp.float32),
            pltpu.SemaphoreType.DMA,
            pltpu.SemaphoreType.DMA,
            pltpu.SemaphoreType.DMA,
            pltpu.SemaphoreType.DMA,
            pltpu.SemaphoreType.DMA,
            pltpu.SemaphoreType.DMA,
        ],
    )
    def emb_kernel(packed_hbm, table_hbm, pos_hbm, out_hbm,
                   packed_v, tok_v, pidx_v, rows_v, prows_v,
                   sgt0, sgt1, sgp0, sgp1, so0, so1):
        wid = lax.axis_index("s") * NC + lax.axis_index("c")
        w_base = wid * rows_per_w
        sgt = (sgt0, sgt1)
        sgp = (sgp0, sgp1)
        so = (so0, so1)
        lane = lax.iota(jnp.int32, LANES)
        rep_off = lane * n_pos_rows

        pltpu.async_copy(packed_hbm.at[pl.ds(w_base, rows_per_w)],
                         packed_v, sgt0)
        pltpu.make_async_copy(packed_hbm.at[pl.ds(w_base, rows_per_w)],
                              packed_v, sgt0).wait()

        def issue_gathers(st, b):
            off = pl.multiple_of(st * STEP, 8)

            @plsc.parallel_loop(0, STEP // LANES, unroll=4)
            def _unpack(g):
                sl = pl.ds(g * LANES, LANES)
                pk = packed_v[pl.ds(off + g * LANES, LANES)]
                tok_v[b, sl] = pk & TOK_MASK
                pidx_v[b, sl] = (pk >> TOK_BITS) + rep_off

            for i in range(ROWS_STEP):
                isl = pl.ds(i * SEQ, SEQ)
                pltpu.async_copy(table_hbm.at[tok_v.at[b, isl]],
                                 rows_v.at[b, i], sgt[b])
                pltpu.async_copy(pos_hbm.at[pidx_v.at[b, isl]],
                                 prows_v.at[b, i], sgp[b])

        def wait_gathers(b):
            for i in range(ROWS_STEP):
                isl = pl.ds(i * SEQ, SEQ)
                pltpu.make_async_copy(table_hbm.at[tok_v.at[b, isl]],
                                     rows_v.at[b, i], sgt[b]).wait()
                pltpu.make_async_copy(pos_hbm.at[pidx_v.at[b, isl]],
                                     prows_v.at[b, i], sgp[b]).wait()

        def wait_out(b):
            pltpu.make_async_copy(rows_v.at[b],
                                  out_hbm.at[pl.ds(0, ROWS_STEP)],
                                  so[b]).wait()

        issue_gathers(0, 0)

        def pair_body(j, carry):
            for b in range(2):
                st = 2 * j + b
                nb = 1 - b

                @pl.when(st + 1 < nsteps)
                def _issue_next():
                    @pl.when(st >= 1)
                    def _drain_out():
                        wait_out(nb)
                    issue_gathers(st + 1, nb)

                wait_gathers(b)

                for i in range(ROWS_STEP):
                    @plsc.parallel_loop(0, SEQ, unroll=8)
                    def _row_body(r):
                        for k in range(EMBED // LANES):
                            sl = pl.ds(k * LANES, LANES)
                            rows_v[b, i, r, sl] = (rows_v[b, i, r, sl]
                                                   + prows_v[b, i, r, sl])

                brow = wid * (batch // NW) + st * ROWS_STEP
                pltpu.async_copy(rows_v.at[b],
                                 out_hbm.at[pl.ds(brow, ROWS_STEP)], so[b])
            return carry

        lax.fori_loop(0, nsteps // 2, pair_body, 0)
        wait_out(0)
        wait_out(1)

    return emb_kernel


def kernel(x, padding_mask, table, pos_enc):
    b, s = x.shape
    n_pos_rows = pos_enc.shape[0] + 1
    s_ids = jnp.arange(s, dtype=jnp.int32)[None, :]
    pp = jnp.where(padding_mask, jnp.int32(n_pos_rows - 1), s_ids)
    packed = (x.astype(jnp.int32) | (pp << TOK_BITS)).reshape(b * s)
    pos_ext = jnp.concatenate(
        [pos_enc.astype(jnp.float32),
         jnp.zeros((1, pos_enc.shape[1]), jnp.float32)], axis=0)
    pos_rep = jnp.tile(pos_ext, (POS_REP, 1))
    return _build(b, n_pos_rows)(packed, table, pos_rep)


# speedup vs baseline: 1.9967x; 1.5580x over previous
import functools

import jax
import jax.numpy as jnp
from jax import lax
from jax.experimental import pallas as pl
from jax.experimental.pallas import tpu as pltpu
from jax.experimental.pallas import tpu_sc as plsc

EMBED = 64
NC = 2
NS = 16
NW = NC * NS

STEP = 800
HALVES = 2


@functools.cache
def _build(ntok):
    rows_per_w = ntok // NW
    nsteps = rows_per_w // STEP
    mesh = plsc.VectorSubcoreMesh(core_axis_name="c", subcore_axis_name="s")

    @functools.partial(
        pl.kernel,
        out_type=jax.ShapeDtypeStruct((ntok, EMBED), jnp.float32),
        mesh=mesh,
        compiler_params=pltpu.CompilerParams(use_tc_tiling_on_sc=False,
                                             needs_layout_passes=False),
        scratch_types=[
            pltpu.VMEM((rows_per_w,), jnp.int32),
            pltpu.VMEM((STEP, EMBED), jnp.float32),
            pltpu.VMEM((STEP, EMBED), jnp.float32),
            pltpu.SemaphoreType.DMA,
            pltpu.SemaphoreType.DMA,
            pltpu.SemaphoreType.DMA,
            pltpu.SemaphoreType.DMA,
        ],
    )
    def gather_kernel(tok_hbm, table_hbm, out_hbm,
                      tok_v, rows0, rows1, sgt0, sgt1, so0, so1):
        wid = lax.axis_index("s") * NC + lax.axis_index("c")
        w_base = wid * rows_per_w
        rows = (rows0, rows1)
        sgt = (sgt0, sgt1)
        so = (so0, so1)

        pltpu.async_copy(tok_hbm.at[pl.ds(w_base, rows_per_w)], tok_v, sgt0)
        pltpu.make_async_copy(tok_hbm.at[pl.ds(w_base, rows_per_w)], tok_v,
                              sgt0).wait()

        def issue_gather(st, b):
            off = pl.multiple_of(st * STEP, 8)
            pltpu.async_copy(table_hbm.at[tok_v.at[pl.ds(off, STEP)]],
                             rows[b], sgt[b])

        def wait_gather(b):
            pltpu.make_async_copy(table_hbm.at[tok_v.at[pl.ds(0, STEP)]],
                                  rows[b], sgt[b]).wait()

        def wait_out(b):
            pltpu.make_async_copy(rows[b], out_hbm.at[pl.ds(0, STEP)],
                                  so[b]).wait()

        issue_gather(0, 0)

        def pair_body(j, carry):
            for b in range(2):
                st = 2 * j + b
                nb = 1 - b

                @pl.when(st + 1 < nsteps)
                def _issue_next():
                    @pl.when(st >= 1)
                    def _drain_out():
                        wait_out(nb)
                    issue_gather(st + 1, nb)

                wait_gather(b)
                base = pl.multiple_of(w_base + st * STEP, 8)
                pltpu.async_copy(rows[b], out_hbm.at[pl.ds(base, STEP)], so[b])
            return carry

        lax.fori_loop(0, nsteps // 2, pair_body, 0)
        wait_out(0)
        wait_out(1)

    return gather_kernel


def kernel(x, padding_mask, table, pos_enc):
    b, s = x.shape
    bh = b // HALVES
    ntok_h = bh * s
    gk = _build(ntok_h)
    pos = pos_enc[None, :s, :].astype(jnp.float32)
    outs = []
    for h in range(HALVES):
        xh = x[h * bh:(h + 1) * bh].reshape(ntok_h).astype(jnp.int32)
        mh = padding_mask[h * bh:(h + 1) * bh]
        g = gk(xh, table).reshape(bh, s, EMBED)
        outs.append(g + jnp.where(mh[..., None], 0.0, pos))
    return jnp.concatenate(outs, axis=0)


# speedup vs baseline: 2.1046x; 1.0541x over previous
import functools

import jax
import jax.numpy as jnp
from jax import lax
from jax.experimental import pallas as pl
from jax.experimental.pallas import tpu as pltpu
from jax.experimental.pallas import tpu_sc as plsc

EMBED = 64
NC = 2
NS = 16
NW = NC * NS

STEP = 800
HALVES = 1


@functools.cache
def _build(ntok):
    rows_per_w = ntok // NW
    nsteps = rows_per_w // STEP
    mesh = plsc.VectorSubcoreMesh(core_axis_name="c", subcore_axis_name="s")

    @functools.partial(
        pl.kernel,
        out_type=jax.ShapeDtypeStruct((ntok, EMBED), jnp.float32),
        mesh=mesh,
        compiler_params=pltpu.CompilerParams(use_tc_tiling_on_sc=False,
                                             needs_layout_passes=False),
        scratch_types=[
            pltpu.VMEM((rows_per_w,), jnp.int32),
            pltpu.VMEM((STEP, EMBED), jnp.float32),
            pltpu.VMEM((STEP, EMBED), jnp.float32),
            pltpu.SemaphoreType.DMA,
            pltpu.SemaphoreType.DMA,
            pltpu.SemaphoreType.DMA,
            pltpu.SemaphoreType.DMA,
        ],
    )
    def gather_kernel(tok_hbm, table_hbm, out_hbm,
                      tok_v, rows0, rows1, sgt0, sgt1, so0, so1):
        wid = lax.axis_index("s") * NC + lax.axis_index("c")
        w_base = wid * rows_per_w
        rows = (rows0, rows1)
        sgt = (sgt0, sgt1)
        so = (so0, so1)

        pltpu.async_copy(tok_hbm.at[pl.ds(w_base, rows_per_w)], tok_v, sgt0)
        pltpu.make_async_copy(tok_hbm.at[pl.ds(w_base, rows_per_w)], tok_v,
                              sgt0).wait()

        def issue_gather(st, b):
            off = pl.multiple_of(st * STEP, 8)
            pltpu.async_copy(table_hbm.at[tok_v.at[pl.ds(off, STEP)]],
                             rows[b], sgt[b])

        def wait_gather(b):
            pltpu.make_async_copy(table_hbm.at[tok_v.at[pl.ds(0, STEP)]],
                                  rows[b], sgt[b]).wait()

        def wait_out(b):
            pltpu.make_async_copy(rows[b], out_hbm.at[pl.ds(0, STEP)],
                                  so[b]).wait()

        issue_gather(0, 0)

        def pair_body(j, carry):
            for b in range(2):
                st = 2 * j + b
                nb = 1 - b

                @pl.when(st + 1 < nsteps)
                def _issue_next():
                    @pl.when(st >= 1)
                    def _drain_out():
                        wait_out(nb)
                    issue_gather(st + 1, nb)

                wait_gather(b)
                base = pl.multiple_of(w_base + st * STEP, 8)
                pltpu.async_copy(rows[b], out_hbm.at[pl.ds(base, STEP)], so[b])
            return carry

        lax.fori_loop(0, nsteps // 2, pair_body, 0)
        wait_out(0)
        wait_out(1)

    return gather_kernel


def kernel(x, padding_mask, table, pos_enc):
    b, s = x.shape
    bh = b // HALVES
    ntok_h = bh * s
    gk = _build(ntok_h)
    pos = pos_enc[None, :s, :].astype(jnp.float32)
    outs = []
    for h in range(HALVES):
        xh = x[h * bh:(h + 1) * bh].reshape(ntok_h).astype(jnp.int32)
        mh = padding_mask[h * bh:(h + 1) * bh]
        g = gk(xh, table).reshape(bh, s, EMBED)
        outs.append(g + jnp.where(mh[..., None], 0.0, pos))
    return jnp.concatenate(outs, axis=0)
